# baseline (speedup 1.0000x reference)
import functools

import jax
import jax.numpy as jnp
import numpy as np
from jax import lax
from jax.experimental import pallas as pl
from jax.experimental.pallas import tpu as pltpu
from jax.experimental.pallas import tpu_sc as plsc

B, R = 2, 256
NUM_ROIS = B * R
PH, PW = 7, 7
NPTS = PH * PW
C = 256
NCH = C // 16

_Q = 0.21875 * 0.21875
T3 = _Q * 0.125
T4 = _Q * 0.5
T5 = _Q * 2.0

def _body(rois_hbm, t2, t3, t4, t5, out_hbm,
          box_v, y0r, y1r, x0r, x1r, wy7r, wx7r, wyp, wxp,
          idxA, idxB, dstA, dstB, idxO, sem):
    nw = 32
    per_w = NUM_ROIS // nw
    wid = lax.axis_index("s") * 2 + lax.axis_index("c")
    base_roi = wid * per_w

    pltpu.sync_copy(rois_hbm.at[pl.ds(base_roi, per_w)], box_v)

    lane = lax.iota(jnp.int32, 16)
    grid = jnp.minimum(lane, 6).astype(jnp.float32) * jnp.float32(1.0 / 6.0)
    sely = [(lane + 16 * k) // 7 for k in range(4)]
    selx = [(lane + 16 * k) % 7 for k in range(4)]
    zeros16 = lane * 0

    def splat(v):
        return jnp.full((16,), v, jnp.int32)

    def build_and_gather(table, S, b_scalar, y1v, x1v, y2v, x2v):
        Sf = float(S - 1)
        ys = (y1v + grid * (y2v - y1v)) * Sf
        xs = (x1v + grid * (x2v - x1v)) * Sf
        y0i = ys.astype(jnp.int32)
        x0i = xs.astype(jnp.int32)
        wy = ys - y0i.astype(jnp.float32)
        wx = xs - x0i.astype(jnp.float32)
        y1i = jnp.minimum(y0i + 1, S - 1)
        x1i = jnp.minimum(x0i + 1, S - 1)
        y0r[...] = y0i
        y1r[...] = y1i
        x0r[...] = x0i
        x1r[...] = x1i
        wy7r[...] = wy
        wx7r[...] = wx
        base = jnp.full((16,), b_scalar * (S * S), jnp.int32)
        for k in range(4):
            iy0 = plsc.load_gather(y0r, [sely[k]])
            iy1 = plsc.load_gather(y1r, [sely[k]])
            ix0 = plsc.load_gather(x0r, [selx[k]])
            ix1 = plsc.load_gather(x1r, [selx[k]])
            wyp[pl.ds(16 * k, 16)] = plsc.load_gather(wy7r, [sely[k]])
            wxp[pl.ds(16 * k, 16)] = plsc.load_gather(wx7r, [selx[k]])
            r0 = base + iy0 * S
            r1 = base + iy1 * S
            idxA[pl.ds(16 * k, 16)] = r0 + ix0
            idxA[pl.ds(64 + 16 * k, 16)] = r0 + ix1
            idxB[pl.ds(16 * k, 16)] = r1 + ix0
            idxB[pl.ds(64 + 16 * k, 16)] = r1 + ix1
        cpA = pltpu.async_copy(table.at[idxA], dstA, sem)
        cpB = pltpu.async_copy(table.at[idxB], dstB, sem)
        cpA.wait()
        cpB.wait()

    def roi_body(j, carry):
        r = base_roi + j
        b_scalar = r // R
        jv = splat(j)
        y1v = plsc.load_gather(box_v, [jv, zeros16])
        x1v = plsc.load_gather(box_v, [jv, zeros16 + 1])
        y2v = plsc.load_gather(box_v, [jv, zeros16 + 2])
        x2v = plsc.load_gather(box_v, [jv, zeros16 + 3])
        t = (y2v - y1v) * (x2v - x1v)
        ts = jnp.max(t)
        lvl = (2 + (ts > T3).astype(jnp.int32) + (ts >= T4).astype(jnp.int32)
               + (ts > T5).astype(jnp.int32))
        for lvl_c, table, S in ((2, t2, 256), (3, t3, 128), (4, t4, 64), (5, t5, 32)):
            @pl.when(lvl == lvl_c)
            def _():
                build_and_gather(table, S, b_scalar, y1v, x1v, y2v, x2v)

        def pt_body(p, c2):
            pv = splat(p)
            wxv = plsc.load_gather(wxp, [pv])
            wyv = plsc.load_gather(wyp, [pv])
            for c in range(NCH):
                sl = pl.ds(16 * c, 16)
                v00 = dstA[p, sl]
                v01 = dstA[64 + p, sl]
                v10 = dstB[p, sl]
                v11 = dstB[64 + p, sl]
                top = v00 + wxv * (v01 - v00)
                bot = v10 + wxv * (v11 - v10)
                dstA[p, sl] = top + wyv * (bot - top)
            return c2

        lax.fori_loop(0, NPTS, pt_body, 0)
        obase = b_scalar * (PH * PW * R - R) + r
        for k, off in ((0, 0), (1, 16), (2, 32), (3, 40)):
            pk = lane + off if k == 3 else lane + 16 * k
            ivec = jnp.full((16,), obase, jnp.int32) + pk * R
            idxO[pl.ds(off if k == 3 else 16 * k, 16)] = jnp.where(
                pk < NPTS, ivec, NUM_ROIS * NPTS)
        pltpu.sync_copy(dstA.at[pl.ds(0, 56)], out_hbm.at[idxO])
        return carry

    lax.fori_loop(0, per_w, roi_body, 0)


@jax.jit
def _run(rois_flat, t2, t3, t4, t5):
    mesh = plsc.VectorSubcoreMesh(core_axis_name="c", subcore_axis_name="s")
    f = pl.kernel(
        _body,
        out_type=jax.ShapeDtypeStruct((NUM_ROIS * NPTS + 8, C), jnp.float32),
        mesh=mesh,
        compiler_params=pltpu.CompilerParams(needs_layout_passes=False),
        scratch_types=[
            pltpu.VMEM((16, 4), jnp.float32),
            pltpu.VMEM((16,), jnp.int32),
            pltpu.VMEM((16,), jnp.int32),
            pltpu.VMEM((16,), jnp.int32),
            pltpu.VMEM((16,), jnp.int32),
            pltpu.VMEM((16,), jnp.float32),
            pltpu.VMEM((16,), jnp.float32),
            pltpu.VMEM((64,), jnp.float32),
            pltpu.VMEM((64,), jnp.float32),
            pltpu.VMEM((128,), jnp.int32),
            pltpu.VMEM((128,), jnp.int32),
            pltpu.VMEM((128, C), jnp.float32),
            pltpu.VMEM((128, C), jnp.float32),
            pltpu.VMEM((56,), jnp.int32),
            pltpu.SemaphoreType.DMA,
        ],
    )
    return f(rois_flat, t2, t3, t4, t5)


def kernel(rois, feat_p2, feat_p3, feat_p4, feat_p5):
    rois_flat = rois.reshape(NUM_ROIS, 4)
    t2 = feat_p2.reshape(-1, C)
    t3 = feat_p3.reshape(-1, C)
    t4 = feat_p4.reshape(-1, C)
    t5 = feat_p5.reshape(-1, C)
    out = _run(rois_flat, t2, t3, t4, t5)
    x = out[:NUM_ROIS * NPTS].reshape(B, PH, PW, R, C)
    return jnp.transpose(x, (0, 3, 1, 2, 4))


# speedup vs baseline: 1.5597x; 1.5597x over previous
import jax
import jax.numpy as jnp
from jax import lax
from jax.experimental import pallas as pl
from jax.experimental.pallas import tpu as pltpu
from jax.experimental.pallas import tpu_sc as plsc

B, R = 2, 256
NUM_ROIS = B * R
PH, PW = 7, 7
NPTS = PH * PW
C = 256
NCH = C // 16

_Q = 0.21875 * 0.21875
T3 = _Q * 0.125
T4 = _Q * 0.5
T5 = _Q * 2.0

HI = 56
NROWS = 112


def _body(rois_hbm, t2, t3, t4, t5, out_hbm,
          box_v, y0r, y1r, x0r, x1r, wy7r, wx7r,
          wyp0, wxp0, wyp1, wxp1,
          idxA0, idxB0, idxA1, idxB1,
          dstA0, dstB0, dstA1, dstB1, sem0, sem1):
    nw = 32
    per_w = NUM_ROIS // nw
    wid = lax.axis_index("s") * 2 + lax.axis_index("c")
    base_roi = wid * per_w

    pltpu.sync_copy(rois_hbm.at[pl.ds(base_roi, per_w)], box_v)

    lane = lax.iota(jnp.int32, 16)
    grid = jnp.minimum(lane, 6).astype(jnp.float32) * jnp.float32(1.0 / 6.0)
    chunk_off = (0, 16, 32, 48, 40)
    sely = [(lane + o) // 7 for o in chunk_off]
    selx = [(lane + o) % 7 for o in chunk_off]
    zeros16 = lane * 0

    bufs = ((idxA0, idxB0, dstA0, dstB0, wyp0, wxp0, sem0),
            (idxA1, idxB1, dstA1, dstB1, wyp1, wxp1, sem1))

    def splat(v):
        return jnp.full((16,), v, jnp.int32)

    def build_fire(j, buf):
        idxA, idxB, dstA, dstB, wyp, wxp, sem = buf
        r = base_roi + j
        b_scalar = r // R
        jv = splat(j)
        y1v = plsc.load_gather(box_v, [jv, zeros16])
        x1v = plsc.load_gather(box_v, [jv, zeros16 + 1])
        y2v = plsc.load_gather(box_v, [jv, zeros16 + 2])
        x2v = plsc.load_gather(box_v, [jv, zeros16 + 3])
        t = (y2v - y1v) * (x2v - x1v)
        ts = jnp.max(t)
        lvl = (2 + (ts > T3).astype(jnp.int32) + (ts >= T4).astype(jnp.int32)
               + (ts > T5).astype(jnp.int32))

        def build(table, S):
            Sf = float(S - 1)
            ys = (y1v + grid * (y2v - y1v)) * Sf
            xs = (x1v + grid * (x2v - x1v)) * Sf
            y0i = ys.astype(jnp.int32)
            x0i = xs.astype(jnp.int32)
            wy = ys - y0i.astype(jnp.float32)
            wx = xs - x0i.astype(jnp.float32)
            y1i = jnp.minimum(y0i + 1, S - 1)
            x1i = jnp.minimum(x0i + 1, S - 1)
            y0r[...] = y0i
            y1r[...] = y1i
            x0r[...] = x0i
            x1r[...] = x1i
            wy7r[...] = wy
            wx7r[...] = wx
            base = jnp.full((16,), b_scalar * (S * S), jnp.int32)
            iy0s, iy1s = [], []
            for k in range(4):
                off = chunk_off[k]
                iy0 = plsc.load_gather(y0r, [sely[k]])
                iy1 = plsc.load_gather(y1r, [sely[k]])
                ix0 = plsc.load_gather(x0r, [selx[k]])
                iy0s.append(iy0)
                iy1s.append(iy1)
                wyp[pl.ds(off, 16)] = plsc.load_gather(wy7r, [sely[k]])
                wxp[pl.ds(off, 16)] = plsc.load_gather(wx7r, [selx[k]])
                idxA[pl.ds(off, 16)] = base + iy0 * S + ix0
                idxB[pl.ds(off, 16)] = base + iy1 * S + ix0
            for k in range(3):
                off = chunk_off[k]
                ix1 = plsc.load_gather(x1r, [selx[k]])
                idxA[pl.ds(HI + off, 16)] = base + iy0s[k] * S + ix1
                idxB[pl.ds(HI + off, 16)] = base + iy1s[k] * S + ix1
            iy0 = plsc.load_gather(y0r, [sely[4]])
            iy1 = plsc.load_gather(y1r, [sely[4]])
            ix1 = plsc.load_gather(x1r, [selx[4]])
            idxA[pl.ds(HI + 40, 16)] = base + iy0 * S + ix1
            idxB[pl.ds(HI + 40, 16)] = base + iy1 * S + ix1
            pltpu.async_copy(table.at[idxA], dstA, sem)
            pltpu.async_copy(table.at[idxB], dstB, sem)

        for lvl_c, table, S in ((2, t2, 256), (3, t3, 128), (4, t4, 64), (5, t5, 32)):
            @pl.when(lvl == lvl_c)
            def _():
                build(table, S)

    def wait_gather(buf):
        _, _, dstA, dstB, _, _, sem = buf
        pltpu.make_async_copy(t2.at[pl.ds(0, NROWS)], dstA, sem).wait()
        pltpu.make_async_copy(t2.at[pl.ds(0, NROWS)], dstB, sem).wait()

    def combine_out(j, buf):
        _, _, dstA, dstB, wyp, wxp, _ = buf
        r = base_roi + j

        def pt_body(p, c2):
            pv = splat(p)
            wxv = plsc.load_gather(wxp, [pv])
            wyv = plsc.load_gather(wyp, [pv])
            for c in range(NCH):
                sl = pl.ds(16 * c, 16)
                v00 = dstA[p, sl]
                v01 = dstA[HI + p, sl]
                v10 = dstB[p, sl]
                v11 = dstB[HI + p, sl]
                top = v00 + wxv * (v01 - v00)
                bot = v10 + wxv * (v11 - v10)
                dstA[p, sl] = top + wyv * (bot - top)
            return c2

        lax.fori_loop(0, NPTS, pt_body, 0)
        pltpu.sync_copy(dstA.at[pl.ds(0, 56)], out_hbm.at[r])

    def pair_body(i, carry):
        j0 = 2 * i
        build_fire(j0, bufs[0])
        build_fire(j0 + 1, bufs[1])
        wait_gather(bufs[0])
        combine_out(j0, bufs[0])
        wait_gather(bufs[1])
        combine_out(j0 + 1, bufs[1])
        return carry

    lax.fori_loop(0, per_w // 2, pair_body, 0)


@jax.jit
def _run(rois_flat, t2, t3, t4, t5):
    mesh = plsc.VectorSubcoreMesh(core_axis_name="c", subcore_axis_name="s")
    f = pl.kernel(
        _body,
        out_type=jax.ShapeDtypeStruct((NUM_ROIS, 56, C), jnp.float32),
        mesh=mesh,
        compiler_params=pltpu.CompilerParams(needs_layout_passes=False),
        scratch_types=[
            pltpu.VMEM((16, 4), jnp.float32),
            pltpu.VMEM((16,), jnp.int32),
            pltpu.VMEM((16,), jnp.int32),
            pltpu.VMEM((16,), jnp.int32),
            pltpu.VMEM((16,), jnp.int32),
            pltpu.VMEM((16,), jnp.float32),
            pltpu.VMEM((16,), jnp.float32),
            pltpu.VMEM((64,), jnp.float32),
            pltpu.VMEM((64,), jnp.float32),
            pltpu.VMEM((64,), jnp.float32),
            pltpu.VMEM((64,), jnp.float32),
            pltpu.VMEM((NROWS,), jnp.int32),
            pltpu.VMEM((NROWS,), jnp.int32),
            pltpu.VMEM((NROWS,), jnp.int32),
            pltpu.VMEM((NROWS,), jnp.int32),
            pltpu.VMEM((NROWS, C), jnp.float32),
            pltpu.VMEM((NROWS, C), jnp.float32),
            pltpu.VMEM((NROWS, C), jnp.float32),
            pltpu.VMEM((NROWS, C), jnp.float32),
            pltpu.SemaphoreType.DMA,
            pltpu.SemaphoreType.DMA,
        ],
    )
    return f(rois_flat, t2, t3, t4, t5)


def kernel(rois, feat_p2, feat_p3, feat_p4, feat_p5):
    rois_flat = rois.reshape(NUM_ROIS, 4)
    t2 = feat_p2.reshape(-1, C)
    t3 = feat_p3.reshape(-1, C)
    t4 = feat_p4.reshape(-1, C)
    t5 = feat_p5.reshape(-1, C)
    out = _run(rois_flat, t2, t3, t4, t5)
    return out[:, :NPTS].reshape(B, R, PH, PW, C)


# speedup vs baseline: 1.9658x; 1.2604x over previous
import jax
import jax.numpy as jnp
from jax import lax
from jax.experimental import pallas as pl
from jax.experimental.pallas import tpu as pltpu
from jax.experimental.pallas import tpu_sc as plsc

B, R = 2, 256
NUM_ROIS = B * R
PH, PW = 7, 7
NPTS = PH * PW
C = 256
NCH = C // 16

_Q = 0.21875 * 0.21875
T3 = _Q * 0.125
T4 = _Q * 0.5
T5 = _Q * 2.0

HI = 56
NROWS = 112


def _body(rois_hbm, t2, t3, t4, t5, out_hbm,
          box_v, y0r, y1r, x0r, x1r, wy7r, wx7r,
          wyp0, wxp0, wyp1, wxp1,
          idxA0, idxB0, idxA1, idxB1,
          dstA0, dstB0, dstA1, dstB1, sem0, sem1):
    nw = 32
    per_w = NUM_ROIS // nw
    wid = lax.axis_index("s") * 2 + lax.axis_index("c")
    base_roi = wid * per_w

    pltpu.sync_copy(rois_hbm.at[pl.ds(base_roi, per_w)], box_v)

    lane = lax.iota(jnp.int32, 16)
    grid = jnp.minimum(lane, 6).astype(jnp.float32) * jnp.float32(1.0 / 6.0)
    chunk_off = (0, 16, 32, 48, 40)
    sely = [(lane + o) // 7 for o in chunk_off]
    selx = [(lane + o) % 7 for o in chunk_off]
    zeros16 = lane * 0

    bufs = ((idxA0, idxB0, dstA0, dstB0, wyp0, wxp0, sem0),
            (idxA1, idxB1, dstA1, dstB1, wyp1, wxp1, sem1))

    def splat(v):
        return jnp.full((16,), v, jnp.int32)

    def build_fire(j, buf):
        idxA, idxB, dstA, dstB, wyp, wxp, sem = buf
        r = base_roi + j
        b_scalar = r // R
        jv = splat(j)
        y1v = plsc.load_gather(box_v, [jv, zeros16])
        x1v = plsc.load_gather(box_v, [jv, zeros16 + 1])
        y2v = plsc.load_gather(box_v, [jv, zeros16 + 2])
        x2v = plsc.load_gather(box_v, [jv, zeros16 + 3])
        t = (y2v - y1v) * (x2v - x1v)
        ts = jnp.max(t)
        lvl = (2 + (ts > T3).astype(jnp.int32) + (ts >= T4).astype(jnp.int32)
               + (ts > T5).astype(jnp.int32))

        def build(table, S):
            Sf = float(S - 1)
            ys = (y1v + grid * (y2v - y1v)) * Sf
            xs = (x1v + grid * (x2v - x1v)) * Sf
            y0i = ys.astype(jnp.int32)
            x0i = xs.astype(jnp.int32)
            wy = ys - y0i.astype(jnp.float32)
            wx = xs - x0i.astype(jnp.float32)
            y1i = jnp.minimum(y0i + 1, S - 1)
            x1i = jnp.minimum(x0i + 1, S - 1)
            y0r[...] = y0i
            y1r[...] = y1i
            x0r[...] = x0i
            x1r[...] = x1i
            wy7r[...] = wy
            wx7r[...] = wx
            base = jnp.full((16,), b_scalar * (S * S), jnp.int32)
            iy0s, iy1s = [], []
            for k in range(4):
                off = chunk_off[k]
                iy0 = plsc.load_gather(y0r, [sely[k]])
                iy1 = plsc.load_gather(y1r, [sely[k]])
                ix0 = plsc.load_gather(x0r, [selx[k]])
                iy0s.append(iy0)
                iy1s.append(iy1)
                wyp[pl.ds(off, 16)] = plsc.load_gather(wy7r, [sely[k]])
                wxp[pl.ds(off, 16)] = plsc.load_gather(wx7r, [selx[k]])
                idxA[pl.ds(off, 16)] = base + iy0 * S + ix0
                idxB[pl.ds(off, 16)] = base + iy1 * S + ix0
            for k in range(3):
                off = chunk_off[k]
                ix1 = plsc.load_gather(x1r, [selx[k]])
                idxA[pl.ds(HI + off, 16)] = base + iy0s[k] * S + ix1
                idxB[pl.ds(HI + off, 16)] = base + iy1s[k] * S + ix1
            iy0 = plsc.load_gather(y0r, [sely[4]])
            iy1 = plsc.load_gather(y1r, [sely[4]])
            ix1 = plsc.load_gather(x1r, [selx[4]])
            idxA[pl.ds(HI + 40, 16)] = base + iy0 * S + ix1
            idxB[pl.ds(HI + 40, 16)] = base + iy1 * S + ix1
            pltpu.async_copy(table.at[idxA], dstA, sem)
            pltpu.async_copy(table.at[idxB], dstB, sem)

        for lvl_c, table, S in ((2, t2, 256), (3, t3, 128), (4, t4, 64), (5, t5, 32)):
            @pl.when(lvl == lvl_c)
            def _():
                build(table, S)

    def wait_gather(buf):
        _, _, dstA, dstB, _, _, sem = buf
        pltpu.make_async_copy(t2.at[pl.ds(0, NROWS)], dstA, sem).wait()
        pltpu.make_async_copy(t2.at[pl.ds(0, NROWS)], dstB, sem).wait()

    def combine_out(j, buf):
        _, _, dstA, dstB, wyp, wxp, _ = buf
        r = base_roi + j

        def pt_body(p, c2):
            pv = splat(p)
            wxv = plsc.load_gather(wxp, [pv])
            wyv = plsc.load_gather(wyp, [pv])
            for c in range(NCH):
                sl = pl.ds(16 * c, 16)
                v00 = dstA[p, sl]
                v01 = dstA[HI + p, sl]
                v10 = dstB[p, sl]
                v11 = dstB[HI + p, sl]
                top = v00 + wxv * (v01 - v00)
                bot = v10 + wxv * (v11 - v10)
                dstA[p, sl] = top + wyv * (bot - top)
            return c2

        lax.fori_loop(0, NPTS, pt_body, 0)
        pltpu.sync_copy(dstA.at[pl.ds(0, 56)], out_hbm.at[r])

    def roi_body(j, carry):
        build_fire(j, bufs[0])
        wait_gather(bufs[0])
        combine_out(j, bufs[0])
        return carry

    lax.fori_loop(0, per_w, roi_body, 0)


@jax.jit
def _run(rois_flat, t2, t3, t4, t5):
    mesh = plsc.VectorSubcoreMesh(core_axis_name="c", subcore_axis_name="s")
    f = pl.kernel(
        _body,
        out_type=jax.ShapeDtypeStruct((NUM_ROIS, 56, C), jnp.float32),
        mesh=mesh,
        compiler_params=pltpu.CompilerParams(needs_layout_passes=False),
        scratch_types=[
            pltpu.VMEM((16, 4), jnp.float32),
            pltpu.VMEM((16,), jnp.int32),
            pltpu.VMEM((16,), jnp.int32),
            pltpu.VMEM((16,), jnp.int32),
            pltpu.VMEM((16,), jnp.int32),
            pltpu.VMEM((16,), jnp.float32),
            pltpu.VMEM((16,), jnp.float32),
            pltpu.VMEM((64,), jnp.float32),
            pltpu.VMEM((64,), jnp.float32),
            pltpu.VMEM((64,), jnp.float32),
            pltpu.VMEM((64,), jnp.float32),
            pltpu.VMEM((NROWS,), jnp.int32),
            pltpu.VMEM((NROWS,), jnp.int32),
            pltpu.VMEM((NROWS,), jnp.int32),
            pltpu.VMEM((NROWS,), jnp.int32),
            pltpu.VMEM((NROWS, C), jnp.float32),
            pltpu.VMEM((NROWS, C), jnp.float32),
            pltpu.VMEM((NROWS, C), jnp.float32),
            pltpu.VMEM((NROWS, C), jnp.float32),
            pltpu.SemaphoreType.DMA,
            pltpu.SemaphoreType.DMA,
        ],
    )
    return f(rois_flat, t2, t3, t4, t5)


def kernel(rois, feat_p2, feat_p3, feat_p4, feat_p5):
    rois_flat = rois.reshape(NUM_ROIS, 4)
    t2 = feat_p2.reshape(-1, C)
    t3 = feat_p3.reshape(-1, C)
    t4 = feat_p4.reshape(-1, C)
    t5 = feat_p5.reshape(-1, C)
    out = _run(rois_flat, t2, t3, t4, t5)
    return out[:, :NPTS].reshape(B, R, PH, PW, C)
